# baseline (speedup 1.0000x reference)
import functools

import jax
import jax.numpy as jnp
from jax import lax
from jax.experimental import pallas as pl
from jax.experimental.pallas import tpu as pltpu
from jax.experimental.pallas import tpu_sc as plsc

N = 10000
H = 128
FW = 64
E = 320000
NC = 2
NS = 16
C = 80
EW = E // NS
KC = EW // C
RPT = N // NS
NSLOT = 9
LK = NSLOT - 2
RPC = -(-RPT // C)


def _make_scatter():
    mesh = plsc.VectorSubcoreMesh(core_axis_name="c", subcore_axis_name="s")

    @functools.partial(
        pl.kernel,
        out_type=(
            jax.ShapeDtypeStruct((N, H), jnp.float32),
            jax.ShapeDtypeStruct((NC, N, FW), jnp.float32),
        ),
        mesh=mesh,
        scratch_types=[
            pltpu.VMEM((EW,), jnp.int32),
            pltpu.VMEM((EW,), jnp.int32),
            [pltpu.VMEM((C, FW), jnp.float32)] * NSLOT,
            pltpu.VMEM_SHARED((N, FW), jnp.float32),
            [pltpu.SemaphoreType.DMA] * NSLOT,
            [pltpu.SemaphoreType.DMA] * NSLOT,
        ],
        compiler_params=pltpu.CompilerParams(use_tc_tiling_on_sc=False),
    )
    def scatter_k(h_hbm, e_hbm, out_hbm, tab_hbm,
                  src_v, dst_v, bufs, acc, gsems, ssems):
        cid = lax.axis_index("c")
        sid = lax.axis_index("s")
        table = tab_hbm.at[cid]

        def src_idx(i):
            return src_v.at[pl.ds(i * C, C)]

        def dst_idx(i):
            return dst_v.at[pl.ds(i * C, C)]

        def g_issue(i, b):
            pltpu.async_copy(table.at[src_idx(i)], bufs[b], gsems[b])

        def g_wait(i, b):
            pltpu.make_async_copy(table.at[src_idx(i)], bufs[b], gsems[b]).wait()

        def s_issue(i, b):
            pltpu.async_copy(bufs[b], acc.at[dst_idx(i)], ssems[b], add=True)

        def s_wait(i, b):
            pltpu.make_async_copy(bufs[b], acc.at[dst_idx(i)], ssems[b]).wait()

        r0 = sid * RPT

        idx_s = pltpu.async_copy(
            e_hbm.at[0, pl.ds(sid * EW, EW)], src_v, gsems[NSLOT - 2])
        idx_d = pltpu.async_copy(
            e_hbm.at[1, pl.ds(sid * EW, EW)], dst_v, gsems[NSLOT - 1])

        @pl.loop(0, C)
        def _(i):
            @pl.loop(0, FW // 16)
            def _(j):
                bufs[2][i, pl.ds(j * 16, 16)] = jnp.zeros((16,), jnp.float32)

        def chunk_rows(j):
            return r0 + j * C, (C if j < RPC - 1 else RPT - (RPC - 1) * C)

        def z_issue(k):
            rr, nr = chunk_rows(k)
            return pltpu.async_copy(
                bufs[2].at[pl.ds(0, nr)], acc.at[pl.ds(rr, nr)], ssems[2])

        def rp_read(j):
            rr, nr = chunk_rows(j)
            return pltpu.async_copy(
                h_hbm.at[pl.ds(rr, nr), pl.ds(cid * FW, FW)],
                bufs[j % 2].at[pl.ds(0, nr)], gsems[j % 2])

        def rp_write(j):
            rr, nr = chunk_rows(j)
            return pltpu.async_copy(
                bufs[j % 2].at[pl.ds(0, nr)],
                table.at[pl.ds(rr, nr)], ssems[j % 2])

        z = z_issue(0)
        rd = {0: rp_read(0), 1: rp_read(1)}
        wr = {}
        zk = 0
        for j in range(RPC):
            rd[j].wait()
            wr[j] = rp_write(j)
            if j >= 1:
                wr[j - 1].wait()
                if j + 1 < RPC:
                    rd[j + 1] = rp_read(j + 1)
            if zk + 1 < RPC:
                z.wait()
                zk += 1
                z = z_issue(zk)
        wr[RPC - 1].wait()
        z.wait()

        idx_s.wait()
        idx_d.wait()

        plsc.subcore_barrier()

        for i in range(LK):
            g_issue(i, i)
        g_wait(0, 0)
        s_issue(0, 0)
        g_issue(LK, LK)
        g_wait(1, 1)
        s_issue(1, 1)
        g_issue(LK + 1, LK + 1)

        n_uniform = KC - LK - 2
        @pl.loop(0, n_uniform // NSLOT)
        def _(k):
            i = NSLOT * k
            for off in range(2, 2 + NSLOT):
                bb = off % NSLOT
                g_wait(i + off, bb)
                s_issue(i + off, bb)
                s_wait(i + off - 2, (off - 2) % NSLOT)
                g_issue(i + off + LK, (off + LK) % NSLOT)

        for i in range(2 + (n_uniform // NSLOT) * NSLOT, KC - LK):
            bb = i % NSLOT
            g_wait(i, bb)
            s_issue(i, bb)
            s_wait(i - 2, (i - 2) % NSLOT)
            g_issue(i + LK, (i + LK) % NSLOT)

        for i in range(KC - LK, KC):
            bb = i % NSLOT
            g_wait(i, bb)
            s_issue(i, bb)
            s_wait(i - 2, (i - 2) % NSLOT)
        s_wait(KC - 2, (KC - 2) % NSLOT)
        s_wait(KC - 1, (KC - 1) % NSLOT)

        plsc.subcore_barrier()

        pltpu.sync_copy(acc.at[pl.ds(r0, RPT)],
                        out_hbm.at[pl.ds(r0, RPT), pl.ds(cid * FW, FW)])

    return scatter_k


_scatter = _make_scatter()

_BN = 1000


def _combine_body(p_ref, h_ref, w_ref, wr_ref, b_ref, o_ref):
    acc = jnp.dot(p_ref[...], w_ref[...], preferred_element_type=jnp.float32)
    acc += jnp.dot(h_ref[...], wr_ref[...], preferred_element_type=jnp.float32)
    o_ref[...] = jnp.maximum(acc + b_ref[...], 0.0)


def _combine(p, h, W, b2, Wr):
    return pl.pallas_call(
        _combine_body,
        grid=(N // _BN,),
        in_specs=[
            pl.BlockSpec((_BN, H), lambda i: (i, 0)),
            pl.BlockSpec((_BN, H), lambda i: (i, 0)),
            pl.BlockSpec((H, H), lambda i: (0, 0)),
            pl.BlockSpec((H, H), lambda i: (0, 0)),
            pl.BlockSpec((1, H), lambda i: (0, 0)),
        ],
        out_specs=pl.BlockSpec((_BN, H), lambda i: (i, 0)),
        out_shape=jax.ShapeDtypeStruct((N, H), jnp.float32),
    )(p, h, W, Wr, b2)


def kernel(x, edge_index, W0, b0, Wr0, W1, b1, Wr1):
    b0r = b0.reshape(1, H)
    b1r = b1.reshape(1, H)

    p0, _ = _scatter(x, edge_index)
    h1 = _combine(p0, x, W0, b0r, Wr0)
    p1, _ = _scatter(h1, edge_index)
    h2 = _combine(p1, h1, W1, b1r, Wr1)
    return h2


# speedup vs baseline: 1.0082x; 1.0082x over previous
import functools

import jax
import jax.numpy as jnp
from jax import lax
from jax.experimental import pallas as pl
from jax.experimental.pallas import tpu as pltpu
from jax.experimental.pallas import tpu_sc as plsc

N = 10000
H = 128
FW = 64
E = 320000
NC = 2
NS = 16
C = 128
EW = E // NS
KCF = EW // C
TAIL = EW - KCF * C
RPT = N // NS


def _make_scatter():
    mesh = plsc.VectorSubcoreMesh(core_axis_name="c", subcore_axis_name="s")

    @functools.partial(
        pl.kernel,
        out_type=(
            jax.ShapeDtypeStruct((N, H), jnp.float32),
            jax.ShapeDtypeStruct((NC, N, FW), jnp.float32),
        ),
        mesh=mesh,
        scratch_types=[
            pltpu.VMEM((EW,), jnp.int32),
            pltpu.VMEM((EW,), jnp.int32),
            pltpu.VMEM((C, FW), jnp.float32),
            pltpu.VMEM((C, FW), jnp.float32),
            pltpu.VMEM((C, FW), jnp.float32),
            pltpu.VMEM((C, FW), jnp.float32),
            pltpu.VMEM((C, FW), jnp.float32),
            pltpu.VMEM((C, FW), jnp.float32),
            pltpu.VMEM_SHARED((N, FW), jnp.float32),
            pltpu.SemaphoreType.DMA,
            pltpu.SemaphoreType.DMA,
            pltpu.SemaphoreType.DMA,
            pltpu.SemaphoreType.DMA,
            pltpu.SemaphoreType.DMA,
            pltpu.SemaphoreType.DMA,
            pltpu.SemaphoreType.DMA,
            pltpu.SemaphoreType.DMA,
            pltpu.SemaphoreType.DMA,
            pltpu.SemaphoreType.DMA,
            pltpu.SemaphoreType.DMA,
            pltpu.SemaphoreType.DMA,
        ],
        compiler_params=pltpu.CompilerParams(use_tc_tiling_on_sc=False),
    )
    def scatter_k(h_hbm, e_hbm, out_hbm, tab_hbm,
                  src_v, dst_v, b0, b1, b2, b3, b4, b5, acc,
                  g0, g1, g2, g3, g4, g5, s0, s1, s2, s3, s4, s5):
        cid = lax.axis_index("c")
        sid = lax.axis_index("s")
        table = tab_hbm.at[cid]
        bufs = (b0, b1, b2, b3, b4, b5)
        gsems = (g0, g1, g2, g3, g4, g5)
        ssems = (s0, s1, s2, s3, s4, s5)

        def src_idx(i):
            return src_v.at[pl.ds(i * C, C)]

        def dst_idx(i):
            return dst_v.at[pl.ds(i * C, C)]

        def g_issue(i, b):
            pltpu.async_copy(table.at[src_idx(i)], bufs[b], gsems[b])

        def g_wait(i, b):
            pltpu.make_async_copy(table.at[src_idx(i)], bufs[b], gsems[b]).wait()

        def s_issue(i, b):
            pltpu.async_copy(bufs[b], acc.at[dst_idx(i)], ssems[b], add=True)

        def s_wait(i, b):
            pltpu.make_async_copy(bufs[b], acc.at[dst_idx(i)], ssems[b]).wait()

        r0 = sid * RPT

        idx_s = pltpu.async_copy(e_hbm.at[0, pl.ds(sid * EW, EW)], src_v, g0)
        idx_d = pltpu.async_copy(e_hbm.at[1, pl.ds(sid * EW, EW)], dst_v, g1)

        @pl.loop(0, C)
        def _(i):
            @pl.loop(0, FW // 16)
            def _(j):
                b1[i, pl.ds(j * 16, 16)] = jnp.zeros((16,), jnp.float32)

        def rp_rows(j):
            rr = r0 + j * C
            nr = C if j < RPT // C else RPT - (RPT // C) * C
            return rr, nr

        def rp_read(j, bb, sem):
            rr, nr = rp_rows(j)
            return pltpu.async_copy(
                h_hbm.at[pl.ds(rr, nr), pl.ds(cid * FW, FW)],
                bb.at[pl.ds(0, nr)], sem)

        def rp_write(j, bb, sem):
            rr, nr = rp_rows(j)
            return pltpu.async_copy(bb.at[pl.ds(0, nr)],
                                    table.at[pl.ds(rr, nr)], sem)

        def z_issue(k):
            if k < 4:
                return pltpu.async_copy(b1, acc.at[pl.ds(r0 + k * C, C)], s1)
            return pltpu.async_copy(
                b1.at[pl.ds(0, RPT - 4 * C)],
                acc.at[pl.ds(r0 + 4 * C, RPT - 4 * C)], s1)

        z = z_issue(0)
        rd0 = rp_read(0, b0, g2)
        rd1 = rp_read(1, b2, g3)
        rd0.wait()
        wr0 = rp_write(0, b0, s0)
        rd1.wait()
        wr1 = rp_write(1, b2, s2)
        z.wait()
        z = z_issue(1)
        wr0.wait()
        rd2 = rp_read(2, b0, g2)
        wr1.wait()
        rd3 = rp_read(3, b2, g3)
        z.wait()
        z = z_issue(2)
        rd2.wait()
        wr2 = rp_write(2, b0, s0)
        rd3.wait()
        wr3 = rp_write(3, b2, s2)
        z.wait()
        z = z_issue(3)
        wr2.wait()
        rd4 = rp_read(4, b0, g2)
        z.wait()
        z = z_issue(4)
        rd4.wait()
        wr3.wait()
        wr4 = rp_write(4, b0, s0)
        wr4.wait()
        z.wait()

        idx_s.wait()
        idx_d.wait()

        plsc.subcore_barrier()

        for i in range(4):
            g_issue(i, i)
        g_wait(0, 0)
        s_issue(0, 0)
        g_issue(4, 4)
        g_wait(1, 1)
        s_issue(1, 1)
        g_issue(5, 5)

        @pl.loop(0, (KCF - 6) // 6)
        def _(k):
            i = 6 * k
            for off in (2, 3, 4, 5, 6, 7):
                bb = off % 6
                g_wait(i + off, bb)
                s_issue(i + off, bb)
                s_wait(i + off - 2, (off - 2) % 6)
                g_issue(i + off + 4, (off + 4) % 6)

        for i in range(KCF - 4, KCF):
            bb = i % 6
            g_wait(i, bb)
            s_issue(i, bb)
            s_wait(i - 2, (i - 2) % 6)
        s_wait(KCF - 2, (KCF - 2) % 6)
        s_wait(KCF - 1, (KCF - 1) % 6)

        t0 = KCF * C
        pltpu.async_copy(
            table.at[src_v.at[pl.ds(t0, TAIL)]], b0.at[pl.ds(0, TAIL)], g0
        ).wait()
        pltpu.async_copy(
            b0.at[pl.ds(0, TAIL)], acc.at[dst_v.at[pl.ds(t0, TAIL)]], s0,
            add=True,
        ).wait()

        plsc.subcore_barrier()

        pltpu.sync_copy(acc.at[pl.ds(r0, RPT)],
                        out_hbm.at[pl.ds(r0, RPT), pl.ds(cid * FW, FW)])

    return scatter_k


_scatter = _make_scatter()

_BN = 1000


def _combine_body(p_ref, h_ref, w_ref, wr_ref, b_ref, o_ref):
    acc = jnp.dot(p_ref[...], w_ref[...], preferred_element_type=jnp.float32)
    acc += jnp.dot(h_ref[...], wr_ref[...], preferred_element_type=jnp.float32)
    o_ref[...] = jnp.maximum(acc + b_ref[...], 0.0)


def _combine(p, h, W, b2, Wr):
    return pl.pallas_call(
        _combine_body,
        grid=(N // _BN,),
        in_specs=[
            pl.BlockSpec((_BN, H), lambda i: (i, 0)),
            pl.BlockSpec((_BN, H), lambda i: (i, 0)),
            pl.BlockSpec((H, H), lambda i: (0, 0)),
            pl.BlockSpec((H, H), lambda i: (0, 0)),
            pl.BlockSpec((1, H), lambda i: (0, 0)),
        ],
        out_specs=pl.BlockSpec((_BN, H), lambda i: (i, 0)),
        out_shape=jax.ShapeDtypeStruct((N, H), jnp.float32),
    )(p, h, W, Wr, b2)


def kernel(x, edge_index, W0, b0, Wr0, W1, b1, Wr1):
    b0r = b0.reshape(1, H)
    b1r = b1.reshape(1, H)

    p0, _ = _scatter(x, edge_index)
    h1 = _combine(p0, x, W0, b0r, Wr0)
    p1, _ = _scatter(h1, edge_index)
    h2 = _combine(p1, h1, W1, b1r, Wr1)
    return h2
